# baseline (speedup 1.0000x reference)
import functools
import numpy as np
import jax
import jax.numpy as jnp
from jax import lax
from jax.experimental import pallas as pl
from jax.experimental.pallas import tpu as pltpu, tpu_sc as plsc

L, C, N, M, BITS = 100, 100, 13000, 200000, 64
NC, NS = 2, 16
NW = NC * NS
CH = 512
CPW = 1
SC_COLS = NW * CPW * CH
W = 25088
NSTEPS = 7
SPLIT = W * NSTEPS
REM = M - SPLIT - SC_COLS
INF32 = np.int32(0x7FFFFFFF)
MINT = np.int32(-2147483648)
BITMASK = [np.uint32(1 << b).astype(np.int32) for b in range(32)]


def _anchor_body(tl_ref, trlT_ref, tcT_ref, aB_ref, keys_ref, rank_ref,
                 oh_ref):
    tl = tl_ref[...].astype(jnp.bfloat16)
    trlT = trlT_ref[...].astype(jnp.bfloat16)
    simd = jax.lax.dot_general(tl, trlT, (((1,), (0,)), ((), ())),
                               preferred_element_type=jnp.float32)
    sim = (simd > 0.0).astype(jnp.bfloat16)
    svote = jax.lax.dot_general(sim, tcT_ref[...].astype(jnp.bfloat16),
                                (((1,), (1,)), ((), ())),
                                preferred_element_type=jnp.float32)
    aB = jnp.sign(svote)
    aB_ref[...] = aB

    neg = (aB < 0.0).astype(jnp.int32)
    pw = jnp.left_shift(jnp.int32(1),
                        lax.broadcasted_iota(jnp.int32, (1, 32), 1))
    klo = jnp.sum(neg[:, :32] * pw, axis=1, keepdims=True)
    khi = jnp.sum(neg[:, 32:] * pw, axis=1, keepdims=True)
    klo = klo ^ MINT
    khi = khi ^ MINT
    valid = (jnp.sum((aB == 0.0).astype(jnp.float32), axis=1,
                     keepdims=True) == 0.0)

    klo_c = jnp.concatenate([klo, jnp.full((128 - L, 1), INF32, jnp.int32)], 0)
    khi_c = jnp.concatenate([khi, jnp.full((128 - L, 1), INF32, jnp.int32)], 0)
    klo_r2 = klo_c.reshape(1, 128)
    khi_r2 = khi_c.reshape(1, 128)
    lt = ((khi_c < khi_r2) |
          ((khi_c == khi_r2) & (klo_c < klo_r2)))
    eqk = (khi_c == khi_r2) & (klo_c == klo_r2)
    jlti = (lax.broadcasted_iota(jnp.int32, (128, 128), 0)
            < lax.broadcasted_iota(jnp.int32, (128, 128), 1))
    rank = jnp.sum((lt | (eqk & jlti)).astype(jnp.float32), axis=0,
                   keepdims=True)
    first_slot = jnp.sum(lt.astype(jnp.float32), axis=0)

    keys_ref[0:1, :] = klo_c.reshape(1, 128)
    keys_ref[1:2, :] = khi_c.reshape(1, 128)
    rank_ref[...] = rank.astype(jnp.int32)

    fs_col = first_slot.reshape(128, 1)[:L]
    slot_iota = lax.broadcasted_iota(jnp.int32, (L, 128), 1).astype(jnp.float32)
    oh = ((fs_col == slot_iota) & valid).astype(jnp.float32)
    oh_ref[...] = oh


def _stream_body(aB_ref, rem_ref, dbT_ref, freq_ref, aBb_ref, buf_ref,
                 acc_ref, sem_ref):
    j = pl.program_id(0)

    def start_copy(slot, blk):
        pltpu.make_async_copy(
            dbT_ref.at[:, pl.ds(pl.multiple_of(blk * W, 128), W)],
            buf_ref.at[slot],
            sem_ref.at[slot],
        ).start()

    def count(aBb, db, width):
        matc = jax.lax.dot_general(aBb, db, (((1,), (0,)), ((), ())),
                                   preferred_element_type=jnp.float32)
        eq = (matc > float(BITS - 2)).astype(jnp.bfloat16)
        ones = jnp.ones((width, 128), jnp.bfloat16)
        return jax.lax.dot_general(eq, ones, (((1,), (0,)), ((), ())),
                                   preferred_element_type=jnp.float32)

    @pl.when(j == 0)
    def _init():
        start_copy(0, 0)
        start_copy(1, 1)
        aBb = aB_ref[...].astype(jnp.bfloat16)
        aBb_ref[...] = aBb
        acc_ref[...] = count(aBb, rem_ref[...].astype(jnp.bfloat16), REM)

    slot = jax.lax.rem(j, 2)
    pltpu.make_async_copy(
        dbT_ref.at[:, pl.ds(pl.multiple_of(j * W, 128), W)],
        buf_ref.at[slot],
        sem_ref.at[slot],
    ).wait()
    db = buf_ref[slot].astype(jnp.bfloat16)
    acc_ref[...] += count(aBb_ref[...], db, W)

    @pl.when(j + 2 < NSTEPS)
    def _next():
        start_copy(slot, j + 2)

    @pl.when(j == NSTEPS - 1)
    def _fini():
        freq_ref[...] = acc_ref[...]


def _sc_body(dbT_hbm, keys_hbm, rank_hbm, out_hbm, kb, rk, slo, shi, cnt,
             buf, sem):
    cid = lax.axis_index("c")
    sid = lax.axis_index("s")
    wid = sid * NC + cid
    base = SPLIT + wid * (CPW * CH)

    pltpu.sync_copy(keys_hbm, kb)
    pltpu.sync_copy(rank_hbm, rk)
    pltpu.make_async_copy(
        dbT_hbm.at[:, pl.ds(base, CH)], buf.at[0], sem.at[0]).start()

    for g in range(8):
        sl = pl.ds(g * 16, 16)
        r16 = rk[0, sl]
        plsc.store_scatter(slo, [r16], kb[0, sl])
        plsc.store_scatter(shi, [r16], kb[1, sl])
        cnt[sl] = jnp.zeros((16,), jnp.float32)

    def group(g, slot):
        sl = pl.ds(g * 16, 16)
        klo = jnp.zeros((16,), jnp.int32)
        khi = jnp.zeros((16,), jnp.int32)
        for b in range(32):
            blo = plsc.bitcast(buf[slot, b, sl], jnp.int32)
            bhi = plsc.bitcast(buf[slot, b + 32, sl], jnp.int32)
            klo = klo | (lax.shift_right_logical(blo, 31 - b) & BITMASK[b])
            khi = khi | (lax.shift_right_logical(bhi, 31 - b) & BITMASK[b])
        klo = klo ^ MINT
        khi = khi ^ MINT
        idx = jnp.zeros((16,), jnp.int32)
        for step in (64, 32, 16, 8, 4, 2, 1):
            cand = idx + step
            probe = cand - 1
            plo = plsc.load_gather(slo, [probe])
            phi = plsc.load_gather(shi, [probe])
            lt = (phi < khi) | ((phi == khi) & (plo < klo))
            idx = jnp.where(lt, cand, idx)
        flo = plsc.load_gather(slo, [idx])
        fhi = plsc.load_gather(shi, [idx])
        m = (flo == klo) & (fhi == khi)
        plsc.addupdate_scatter(cnt, [idx], jnp.ones((16,), jnp.float32),
                               mask=m)

    for c in range(CPW):
        slot = c % 2
        if c + 1 < CPW:
            pltpu.make_async_copy(
                dbT_hbm.at[:, pl.ds(base + (c + 1) * CH, CH)],
                buf.at[1 - slot], sem.at[1 - slot]).start()
        pltpu.make_async_copy(
            dbT_hbm.at[:, pl.ds(base + c * CH, CH)],
            buf.at[slot], sem.at[slot]).wait()

        def body(g, _):
            group(2 * g, slot)
            group(2 * g + 1, slot)
            return 0
        lax.fori_loop(0, CH // 32, body, 0)

    pltpu.sync_copy(cnt, out_hbm.at[wid])


def _merge_body(freq_ref, oh_ref, cnt_ref, avg_ref, zero_ref):
    cnt_tot = jnp.sum(cnt_ref[...], axis=0, keepdims=True)
    freq_sc = jax.lax.dot_general(oh_ref[...], cnt_tot,
                                  (((1,), (1,)), ((), ())),
                                  preferred_element_type=jnp.float32)
    freq = freq_ref[...][:, 0:1] + freq_sc
    avg_ref[...] = (jnp.sum(freq) / float(L)).reshape(1, 1)
    zero_ref[...] = jnp.sum((freq == 0.0).astype(jnp.float32)).reshape(1, 1)


def kernel(traincodes, dB, target_labels, trainlabels):
    trlT = trainlabels.T
    tcT = traincodes.T
    dBT = dB.T
    rem = dBT[:, SPLIT + SC_COLS:]

    aB, keys, rank, oh = pl.pallas_call(
        _anchor_body,
        grid=(1,),
        in_specs=[
            pl.BlockSpec((L, C), lambda j: (0, 0)),
            pl.BlockSpec((C, N), lambda j: (0, 0)),
            pl.BlockSpec((BITS, N), lambda j: (0, 0)),
        ],
        out_specs=[
            pl.BlockSpec((L, BITS), lambda j: (0, 0)),
            pl.BlockSpec((2, 128), lambda j: (0, 0)),
            pl.BlockSpec((1, 128), lambda j: (0, 0)),
            pl.BlockSpec((L, 128), lambda j: (0, 0)),
        ],
        out_shape=[
            jax.ShapeDtypeStruct((L, BITS), jnp.float32),
            jax.ShapeDtypeStruct((2, 128), jnp.int32),
            jax.ShapeDtypeStruct((1, 128), jnp.int32),
            jax.ShapeDtypeStruct((L, 128), jnp.float32),
        ],
    )(target_labels, trlT, tcT)

    freqB = pl.pallas_call(
        _stream_body,
        grid=(NSTEPS,),
        in_specs=[
            pl.BlockSpec((L, BITS), lambda j: (0, 0)),
            pl.BlockSpec((BITS, REM), lambda j: (0, 0)),
            pl.BlockSpec(memory_space=pl.ANY),
        ],
        out_specs=pl.BlockSpec((L, 128), lambda j: (0, 0)),
        out_shape=jax.ShapeDtypeStruct((L, 128), jnp.float32),
        scratch_shapes=[
            pltpu.VMEM((L, BITS), jnp.bfloat16),
            pltpu.VMEM((2, BITS, W), jnp.float32),
            pltpu.VMEM((L, 128), jnp.float32),
            pltpu.SemaphoreType.DMA((2,)),
        ],
    )(aB, rem, dBT)

    mesh = plsc.VectorSubcoreMesh(core_axis_name="c", subcore_axis_name="s",
                                  num_cores=NC, num_subcores=NS)
    sc_kernel = functools.partial(
        pl.kernel,
        out_type=jax.ShapeDtypeStruct((NW, 128), jnp.float32),
        mesh=mesh,
        scratch_types=[
            pltpu.VMEM((2, 128), jnp.int32),
            pltpu.VMEM((1, 128), jnp.int32),
            pltpu.VMEM((128,), jnp.int32),
            pltpu.VMEM((128,), jnp.int32),
            pltpu.VMEM((128,), jnp.float32),
            pltpu.VMEM((2, BITS, CH), jnp.float32),
            pltpu.SemaphoreType.DMA((2,)),
        ],
        compiler_params=pltpu.CompilerParams(use_tc_tiling_on_sc=True,
                                             needs_layout_passes=False),
    )(_sc_body)
    counts = sc_kernel(dBT, keys, rank)

    avg, zero = pl.pallas_call(
        _merge_body,
        grid=(1,),
        in_specs=[
            pl.BlockSpec((L, 128), lambda j: (0, 0)),
            pl.BlockSpec((L, 128), lambda j: (0, 0)),
            pl.BlockSpec((NW, 128), lambda j: (0, 0)),
        ],
        out_specs=[
            pl.BlockSpec((1, 1), lambda j: (0, 0)),
            pl.BlockSpec((1, 1), lambda j: (0, 0)),
        ],
        out_shape=[
            jax.ShapeDtypeStruct((1, 1), jnp.float32),
            jax.ShapeDtypeStruct((1, 1), jnp.float32),
        ],
    )(freqB, oh, counts)
    return (aB, avg[0, 0], zero[0, 0])


# speedup vs baseline: 1.4002x; 1.4002x over previous
import jax
import jax.numpy as jnp
from jax.experimental import pallas as pl
from jax.experimental.pallas import tpu as pltpu

L, C, N, M, BITS = 100, 100, 13000, 200000, 64
W = 12800
NSTEPS = 15
REM = M - W * NSTEPS


def _count(aBb, db, width):
    matc = jax.lax.dot_general(aBb, db, (((1,), (0,)), ((), ())),
                               preferred_element_type=jnp.float32)
    eq = (matc > float(BITS - 2)).astype(jnp.bfloat16)
    ones = jnp.ones((width, 128), jnp.bfloat16)
    return jax.lax.dot_general(eq, ones, (((1,), (0,)), ((), ())),
                               preferred_element_type=jnp.float32)


def _body(tl_ref, trlT_ref, tcT_ref, rem_ref, dbT_ref, aB_ref, avg_ref,
          zero_ref, aBb_ref, buf_ref, freq_ref, sem_ref):
    j = pl.program_id(0)

    def start_copy(slot, blk):
        pltpu.make_async_copy(
            dbT_ref.at[:, pl.ds(pl.multiple_of(blk * W, 128), W)],
            buf_ref.at[slot],
            sem_ref.at[slot],
        ).start()

    @pl.when(j == 0)
    def _init():
        start_copy(0, 0)
        start_copy(1, 1)
        tl = tl_ref[...].astype(jnp.bfloat16)
        trlT = trlT_ref[...].astype(jnp.bfloat16)
        simd = jax.lax.dot_general(tl, trlT, (((1,), (0,)), ((), ())),
                                   preferred_element_type=jnp.float32)
        sim = (simd > 0.0).astype(jnp.bfloat16)
        svote = jax.lax.dot_general(sim, tcT_ref[...].astype(jnp.bfloat16),
                                    (((1,), (1,)), ((), ())),
                                    preferred_element_type=jnp.float32)
        aB = jnp.sign(svote)
        aB_ref[...] = aB
        aBb = aB.astype(jnp.bfloat16)
        aBb_ref[...] = aBb
        freq_ref[...] = _count(aBb, rem_ref[...].astype(jnp.bfloat16), REM)

    slot = jax.lax.rem(j, 2)
    pltpu.make_async_copy(
        dbT_ref.at[:, pl.ds(pl.multiple_of(j * W, 128), W)],
        buf_ref.at[slot],
        sem_ref.at[slot],
    ).wait()
    db = buf_ref[slot].astype(jnp.bfloat16)
    freq_ref[...] += _count(aBb_ref[...], db, W)

    @pl.when(j + 2 < NSTEPS)
    def _next():
        start_copy(slot, j + 2)

    @pl.when(j == NSTEPS - 1)
    def _fini():
        freq = freq_ref[...][:, 0:1]
        avg_ref[...] = (jnp.sum(freq) / float(L)).reshape(1, 1)
        zero_ref[...] = jnp.sum((freq == 0.0).astype(jnp.float32)).reshape(1, 1)


def kernel(traincodes, dB, target_labels, trainlabels):
    trlT = trainlabels.T
    tcT = traincodes.T
    dBT = dB.T
    rem = dBT[:, W * NSTEPS:]

    aB, avg, zero = pl.pallas_call(
        _body,
        grid=(NSTEPS,),
        in_specs=[
            pl.BlockSpec((L, C), lambda j: (0, 0)),
            pl.BlockSpec((C, N), lambda j: (0, 0)),
            pl.BlockSpec((BITS, N), lambda j: (0, 0)),
            pl.BlockSpec((BITS, REM), lambda j: (0, 0)),
            pl.BlockSpec(memory_space=pl.ANY),
        ],
        out_specs=[
            pl.BlockSpec((L, BITS), lambda j: (0, 0)),
            pl.BlockSpec((1, 1), lambda j: (0, 0)),
            pl.BlockSpec((1, 1), lambda j: (0, 0)),
        ],
        out_shape=[
            jax.ShapeDtypeStruct((L, BITS), jnp.float32),
            jax.ShapeDtypeStruct((1, 1), jnp.float32),
            jax.ShapeDtypeStruct((1, 1), jnp.float32),
        ],
        scratch_shapes=[
            pltpu.VMEM((L, BITS), jnp.bfloat16),
            pltpu.VMEM((2, BITS, W), jnp.float32),
            pltpu.VMEM((L, 128), jnp.float32),
            pltpu.SemaphoreType.DMA((2,)),
        ],
    )(target_labels, trlT, tcT, rem, dBT)
    return (aB, avg[0, 0], zero[0, 0])
